# Initial kernel scaffold; baseline (speedup 1.0000x reference)
import jax
import jax.numpy as jnp
from jax.experimental import pallas as pl


def kernel(h, pos, edge_attr, We1, be1, We2, be2, Wc1, bc1, Wc2, bc2, Wn, bn, gamma, beta, edge_index):
    raise NotImplementedError("write your pallas kernel here")


# speedup vs baseline: 3.1882x; 3.1882x over previous
import functools

import jax
import jax.numpy as jnp
from jax import lax
from jax.experimental import pallas as pl
from jax.experimental.pallas import tpu as pltpu
from jax.experimental.pallas import tpu_sc as plsc

_NC = 2
_NS = 16
_NW = _NC * _NS
_L = 16

_CH = 128
_PW = 4

_SC_PARAMS = pltpu.CompilerParams(needs_layout_passes=False)


def _build_tables(h, WaT, WbT, be1):
    N, H = h.shape
    Bn = 1000

    def body(h_ref, wa_ref, wb_ref, b1_ref, ta_ref, tb_ref):
        hblk = h_ref[...]
        ta_ref[...] = (
            jnp.dot(hblk, wa_ref[...], preferred_element_type=jnp.float32)
            + b1_ref[...]
        )
        tb_ref[...] = jnp.dot(hblk, wb_ref[...], preferred_element_type=jnp.float32)

    return pl.pallas_call(
        body,
        grid=(N // Bn,),
        in_specs=[
            pl.BlockSpec((Bn, H), lambda i: (i, 0)),
            pl.BlockSpec((H, H), lambda i: (0, 0)),
            pl.BlockSpec((H, H), lambda i: (0, 0)),
            pl.BlockSpec((1, H), lambda i: (0, 0)),
        ],
        out_specs=[
            pl.BlockSpec((Bn, H), lambda i: (i, 0)),
            pl.BlockSpec((Bn, H), lambda i: (i, 0)),
        ],
        out_shape=[
            jax.ShapeDtypeStruct((N, H), jnp.float32),
            jax.ShapeDtypeStruct((N, H), jnp.float32),
        ],
    )(h, WaT, WbT, be1)


def _sc_gather(tA, tB, src_idx, dst_idx, pos4f, we1c):
    N, H = tA.shape
    E = src_idx.shape[0]
    NCH = E // _CH
    n_full = NCH // _NW
    n_extra = NCH % _NW
    NSL = H // _L
    mesh = plsc.VectorSubcoreMesh(core_axis_name="c", subcore_axis_name="s")

    @functools.partial(
        pl.kernel,
        out_type=(
            jax.ShapeDtypeStruct((E, H), jnp.float32),
            jax.ShapeDtypeStruct((E,), jnp.float32),
            jax.ShapeDtypeStruct((3 * E,), jnp.float32),
        ),
        mesh=mesh,
        compiler_params=_SC_PARAMS,
        scratch_types=[
            pltpu.VMEM((_CH,), jnp.int32),
            pltpu.VMEM((_CH,), jnp.int32),
            pltpu.VMEM((_CH, H), jnp.float32),
            pltpu.VMEM((_CH, H), jnp.float32),
            pltpu.VMEM((N * _PW,), jnp.float32),
            pltpu.VMEM((H,), jnp.float32),
            pltpu.VMEM((_CH,), jnp.float32),
            pltpu.VMEM((3 * _CH,), jnp.float32),
            pltpu.SemaphoreType.DMA,
            pltpu.SemaphoreType.DMA,
        ],
    )
    def k(ta_hbm, tb_hbm, si_hbm, di_hbm, p4_hbm, wc_hbm,
          g_hbm, d2_hbm, df_hbm,
          si_v, di_v, bufa, bufb, posb, wb, d2b, dfb, sem_a, sem_b):
        c = lax.axis_index("c")
        s = lax.axis_index("s")
        wid = s * _NC + c
        pltpu.sync_copy(p4_hbm, posb)
        pltpu.sync_copy(wc_hbm, wb)

        def do_chunk(ci):
            off = pl.multiple_of(ci * _CH, _CH)
            pltpu.sync_copy(si_hbm.at[pl.ds(off, _CH)], si_v)
            pltpu.sync_copy(di_hbm.at[pl.ds(off, _CH)], di_v)
            cpa = pltpu.async_copy(ta_hbm.at[si_v], bufa, sem_a)
            cpb = pltpu.async_copy(tb_hbm.at[di_v], bufb, sem_b)
            for gg in range(_CH // _L):
                b16 = gg * _L
                sv = si_v[pl.ds(b16, _L)] * _PW
                dv = di_v[pl.ds(b16, _L)] * _PW
                dx = plsc.load_gather(posb, [dv]) - plsc.load_gather(posb, [sv])
                dy = (plsc.load_gather(posb, [dv + 1])
                      - plsc.load_gather(posb, [sv + 1]))
                dz = (plsc.load_gather(posb, [dv + 2])
                      - plsc.load_gather(posb, [sv + 2]))
                d2 = jnp.minimum(dx * dx + dy * dy + dz * dz, 1000.0)
                d2b[pl.ds(b16, _L)] = d2
                dfb[pl.ds(b16, _L)] = dx
                dfb[pl.ds(_CH + b16, _L)] = dy
                dfb[pl.ds(2 * _CH + b16, _L)] = dz
            cpa.wait()
            cpb.wait()
            wv = [wb[pl.ds(j * _L, _L)] for j in range(NSL)]

            def rowbody(r, carry):
                sd2 = plsc.load_gather(d2b, [jnp.full((_L,), r, jnp.int32)])
                for j in range(NSL):
                    sl = pl.ds(j * _L, _L)
                    bufa[r, sl] = bufa[r, sl] + bufb[r, sl] + sd2 * wv[j]
                return carry

            lax.fori_loop(0, _CH, rowbody, 0)
            pltpu.sync_copy(bufa, g_hbm.at[pl.ds(off, _CH)])
            pltpu.sync_copy(d2b, d2_hbm.at[pl.ds(off, _CH)])
            for q in range(3):
                pltpu.sync_copy(dfb.at[pl.ds(q * _CH, _CH)],
                                df_hbm.at[pl.ds(q * E + off, _CH)])

        def chunk(i, carry):
            do_chunk(wid + i * _NW)
            return carry

        lax.fori_loop(0, n_full, chunk, 0)
        if n_extra:
            @pl.when(wid < n_extra)
            def _():
                do_chunk(wid + n_full * _NW)

    return k(tA, tB, src_idx, dst_idx, pos4f, we1c)


def _edge_mlp(g, edge_attr, WdT, We2T, be2, Wc1sT, bc1s, sgn):
    E, H = g.shape
    DE = edge_attr.shape[1]
    B = 2560

    def body(g_ref, ea_ref, wd_ref, w2_ref, b2_ref, wc1_ref, bc1_ref,
             sg_ref, m_ref, c_ref):
        m1 = g_ref[...] + jnp.dot(
            ea_ref[...], wd_ref[...], preferred_element_type=jnp.float32)
        m1 = jnp.maximum(m1, 0.0)
        m = jnp.maximum(
            jnp.dot(m1, w2_ref[...], preferred_element_type=jnp.float32)
            + b2_ref[...], 0.0)
        c1 = jnp.maximum(
            jnp.dot(m, wc1_ref[...], preferred_element_type=jnp.float32)
            + bc1_ref[...], 0.0)
        m_ref[...] = m
        c_ref[...] = c1 * sg_ref[...]

    return pl.pallas_call(
        body,
        grid=(E // B,),
        in_specs=[
            pl.BlockSpec((B, H), lambda i: (i, 0)),
            pl.BlockSpec((B, DE), lambda i: (i, 0)),
            pl.BlockSpec((DE, H), lambda i: (0, 0)),
            pl.BlockSpec((H, H), lambda i: (0, 0)),
            pl.BlockSpec((1, H), lambda i: (0, 0)),
            pl.BlockSpec((H, H), lambda i: (0, 0)),
            pl.BlockSpec((1, H), lambda i: (0, 0)),
            pl.BlockSpec((1, H), lambda i: (0, 0)),
        ],
        out_specs=[
            pl.BlockSpec((B, H), lambda i: (i, 0)),
            pl.BlockSpec((B, H), lambda i: (i, 0)),
        ],
        out_shape=[
            jax.ShapeDtypeStruct((E, H), jnp.float32),
            jax.ShapeDtypeStruct((E, H), jnp.float32),
        ],
    )(g, edge_attr, WdT, We2T, be2, Wc1sT, bc1s, sgn)


def _sc_scatter(m_arr, c_arr, dst_idx, d2f, diff3, bc2v, zM, zT):
    E, H = m_arr.shape
    N = zM.shape[0]
    NCH = E // _CH
    n_full = NCH // _NW
    n_extra = NCH % _NW
    NRA = (N // _NS) // 8 * 8
    NRT = N - (_NS - 1) * NRA
    _NWR = 8
    WSL = N * _PW // _NWR
    NSL = H // _L
    mesh = plsc.VectorSubcoreMesh(core_axis_name="c", subcore_axis_name="s")

    @functools.partial(
        pl.kernel,
        out_type=(
            jax.ShapeDtypeStruct((_NC * N, H), jnp.float32),
            jax.ShapeDtypeStruct((_NC * N * _PW,), jnp.float32),
        ),
        mesh=mesh,
        compiler_params=_SC_PARAMS,
        scratch_types=[
            pltpu.VMEM((_CH,), jnp.int32),
            pltpu.VMEM((_CH, H), jnp.float32),
            pltpu.VMEM((_CH, H), jnp.float32),
            pltpu.VMEM((_CH,), jnp.float32),
            pltpu.VMEM((3 * _CH,), jnp.float32),
            pltpu.VMEM((_CH * _L,), jnp.float32),
            pltpu.VMEM((3 * _CH,), jnp.float32),
            pltpu.VMEM((_CH,), jnp.int32),
            pltpu.VMEM((_CH,), jnp.int32),
            pltpu.VMEM((_CH,), jnp.int32),
            pltpu.VMEM((_L,), jnp.float32),
            pltpu.VMEM((N * _PW // 8,), jnp.float32),
            pltpu.VMEM_SHARED((N, H), jnp.float32),
            pltpu.VMEM_SHARED((N * _PW,), jnp.float32),
        ],
    )
    def k(m_hbm, c_hbm, di_hbm, d2_hbm, df_hbm, bc2_hbm, zm_hbm, zt_hbm,
          om_hbm, ot_hbm,
          di_v, mbuf, cbuf, d2b, dfb, scob, tbuf, ix0, ix1, ix2, bcb,
          twb, accm, acct):
        c = lax.axis_index("c")
        s = lax.axis_index("s")
        wid = s * _NC + c
        rs = pl.multiple_of(s * NRA, 8)
        pltpu.sync_copy(bc2_hbm, bcb)

        @pl.when(s < _NS - 1)
        def _():
            pltpu.sync_copy(zm_hbm.at[pl.ds(rs, NRA)], accm.at[pl.ds(rs, NRA)])

        @pl.when(s == _NS - 1)
        def _():
            pltpu.sync_copy(zm_hbm.at[pl.ds(rs, NRT)], accm.at[pl.ds(rs, NRT)])

        ws = pl.multiple_of(s * WSL, 8)

        @pl.when(s < _NWR)
        def _():
            pltpu.sync_copy(zt_hbm.at[pl.ds(ws, WSL)], twb)
            pltpu.sync_copy(twb, acct.at[pl.ds(ws, WSL)])

        plsc.subcore_barrier()

        def do_chunk(ci):
            off = pl.multiple_of(ci * _CH, _CH)
            pltpu.sync_copy(di_hbm.at[pl.ds(off, _CH)], di_v)
            pltpu.sync_copy(m_hbm.at[pl.ds(off, _CH)], mbuf)
            pltpu.sync_copy(c_hbm.at[pl.ds(off, _CH)], cbuf)
            pltpu.sync_copy(d2_hbm.at[pl.ds(off, _CH)], d2b)
            for q in range(3):
                pltpu.sync_copy(df_hbm.at[pl.ds(q * E + off, _CH)],
                                dfb.at[pl.ds(q * _CH, _CH)])

            def rowbody(r, carry):
                acc = cbuf[r, pl.ds(0, _L)]
                for j in range(1, NSL):
                    acc = acc + cbuf[r, pl.ds(j * _L, _L)]
                scob[pl.ds(r * _L, _L)] = acc
                return carry

            lax.fori_loop(0, _CH, rowbody, 0)
            bcv = bcb[...]
            for gg in range(_CH // _L):
                b16 = gg * _L
                sl = pl.ds(b16, _L)
                ev0 = (lax.iota(jnp.int32, _L) + b16) * _L
                sco = plsc.load_gather(scob, [ev0])
                for l in range(1, _L):
                    sco = sco + plsc.load_gather(scob, [ev0 + l])
                sco = sco + bcv
                cc = 1.0 - 2.0 / (jnp.exp(2.0 * sco) + 1.0)
                x = d2b[sl] + 1e-8
                y = plsc.bitcast(
                    0x5F3759DF - (plsc.bitcast(x, jnp.int32) >> 1), jnp.float32)
                y = y * (1.5 - 0.5 * x * y * y)
                y = y * (1.5 - 0.5 * x * y * y)
                y = y * (1.5 - 0.5 * x * y * y)
                cf = cc * 0.1 * y
                tbuf[sl] = dfb[sl] * cf
                tbuf[pl.ds(_CH + b16, _L)] = dfb[pl.ds(_CH + b16, _L)] * cf
                tbuf[pl.ds(2 * _CH + b16, _L)] = dfb[pl.ds(2 * _CH + b16, _L)] * cf
                dv4 = di_v[sl] * _PW
                ix0[sl] = dv4
                ix1[sl] = dv4 + 1
                ix2[sl] = dv4 + 2
            pltpu.sync_copy(mbuf, accm.at[di_v], add=True)
            pltpu.sync_copy(tbuf.at[pl.ds(0, _CH)], acct.at[ix0], add=True)
            pltpu.sync_copy(tbuf.at[pl.ds(_CH, _CH)], acct.at[ix1], add=True)
            pltpu.sync_copy(tbuf.at[pl.ds(2 * _CH, _CH)], acct.at[ix2], add=True)

        def chunk(i, carry):
            do_chunk(wid + i * _NW)
            return carry

        lax.fori_loop(0, n_full, chunk, 0)
        if n_extra:
            @pl.when(wid < n_extra)
            def _():
                do_chunk(wid + n_full * _NW)
        plsc.subcore_barrier()
        orow = pl.multiple_of(c * N + rs, 8)

        @pl.when(s < _NS - 1)
        def _():
            pltpu.sync_copy(accm.at[pl.ds(rs, NRA)], om_hbm.at[pl.ds(orow, NRA)])

        @pl.when(s == _NS - 1)
        def _():
            pltpu.sync_copy(accm.at[pl.ds(rs, NRT)], om_hbm.at[pl.ds(orow, NRT)])

        ow = pl.multiple_of(c * (N * _PW) + ws, 8)

        @pl.when(s < _NWR)
        def _():
            pltpu.sync_copy(acct.at[pl.ds(ws, WSL)], twb)
            pltpu.sync_copy(twb, ot_hbm.at[pl.ds(ow, WSL)])

    return k(m_arr, c_arr, dst_idx, d2f, diff3, bc2v, zM, zT)


def _node_update(h, pos4, mparts, tparts, WnaT, WnbT, bn, gamma, beta):
    N, H = h.shape
    Bn = 1000
    nb = N // Bn

    def body(h_ref, p_ref, m0_ref, m1_ref, t0_ref, t1_ref, wa_ref, wb_ref,
             bn_ref, g_ref, b_ref, ho_ref, po_ref):
        agg = m0_ref[...] + m1_ref[...]
        tsum = t0_ref[...] + t1_ref[...]
        hb = h_ref[...]
        hu = (
            jnp.dot(hb, wa_ref[...], preferred_element_type=jnp.float32)
            + jnp.dot(agg, wb_ref[...], preferred_element_type=jnp.float32)
            + bn_ref[...]
        )
        y = hb + jnp.maximum(hu, 0.0)
        mu = jnp.mean(y, axis=1, keepdims=True)
        yc = y - mu
        var = jnp.mean(yc * yc, axis=1, keepdims=True)
        ho_ref[...] = yc * lax.rsqrt(var + 1e-5) * g_ref[...] + b_ref[...]
        po_ref[...] = p_ref[...] + tsum

    return pl.pallas_call(
        body,
        grid=(nb,),
        in_specs=[
            pl.BlockSpec((Bn, H), lambda i: (i, 0)),
            pl.BlockSpec((Bn, _PW), lambda i: (i, 0)),
            pl.BlockSpec((Bn, H), lambda i: (i, 0)),
            pl.BlockSpec((Bn, H), lambda i: (nb + i, 0)),
            pl.BlockSpec((Bn, _PW), lambda i: (i, 0)),
            pl.BlockSpec((Bn, _PW), lambda i: (nb + i, 0)),
            pl.BlockSpec((H, H), lambda i: (0, 0)),
            pl.BlockSpec((H, H), lambda i: (0, 0)),
            pl.BlockSpec((1, H), lambda i: (0, 0)),
            pl.BlockSpec((1, H), lambda i: (0, 0)),
            pl.BlockSpec((1, H), lambda i: (0, 0)),
        ],
        out_specs=[
            pl.BlockSpec((Bn, H), lambda i: (i, 0)),
            pl.BlockSpec((Bn, _PW), lambda i: (i, 0)),
        ],
        out_shape=[
            jax.ShapeDtypeStruct((N, H), jnp.float32),
            jax.ShapeDtypeStruct((N, _PW), jnp.float32),
        ],
    )(h, pos4, mparts, mparts, tparts, tparts, WnaT, WnbT, bn, gamma, beta)


def kernel(h, pos, edge_attr, We1, be1, We2, be2, Wc1, bc1, Wc2, bc2,
           Wn, bn, gamma, beta, edge_index):
    N, H = h.shape
    E = edge_index.shape[1]
    src = edge_index[0]
    dst = edge_index[1]
    pos4 = jnp.pad(pos, ((0, 0), (0, _PW - pos.shape[1])))
    pos4f = pos4.reshape(-1)

    WaT = We1[:, :H].T
    WbT = We1[:, H:2 * H].T
    we1c = We1[:, 2 * H]
    WdT = We1[:, 2 * H + 1:].T

    wabs = jnp.abs(Wc2[0])
    Wc1sT = (Wc1 * wabs[:, None]).T
    bc1s = (bc1 * wabs).reshape(1, H)
    sgn = jnp.sign(Wc2[0]).reshape(1, H)
    bc2v = jnp.broadcast_to(bc2, (_L,))

    tA, tB = _build_tables(h, WaT, WbT, be1.reshape(1, H))
    g, d2f, diff3 = _sc_gather(tA, tB, src, dst, pos4f, we1c)
    m_arr, c_arr = _edge_mlp(
        g, edge_attr, WdT, We2.T, be2.reshape(1, H), Wc1sT, bc1s, sgn)
    zM = jnp.zeros((N, H), jnp.float32)
    zT = jnp.zeros((N * _PW,), jnp.float32)
    mparts, tparts1d = _sc_scatter(m_arr, c_arr, dst, d2f, diff3, bc2v, zM, zT)
    tparts = tparts1d.reshape(_NC * N, _PW)
    h_out, pos4_out = _node_update(
        h, pos4, mparts, tparts, Wn[:, :H].T, Wn[:, H:].T,
        bn.reshape(1, H), gamma.reshape(1, H), beta.reshape(1, H))
    return (h_out, pos4_out[:, :pos.shape[1]])
